# Initial kernel scaffold; baseline (speedup 1.0000x reference)
import jax
import jax.numpy as jnp
from jax.experimental import pallas as pl


def kernel(stu_id, input_exercise, knowledge_masks, student_emb, k_difficulty, e_discrimination, e_k_prob, W1, b1, W2, b2, W3, b3):
    raise NotImplementedError("write your pallas kernel here")


# speedup vs baseline: 2.7783x; 2.7783x over previous
import functools

import jax
import jax.numpy as jnp
from jax import lax
from jax.experimental import pallas as pl
from jax.experimental.pallas import tpu as pltpu
from jax.experimental.pallas import tpu_sc as plsc

_K = 197
_KP = 256
_DISC_COL = _KP + _K
_D = 2 * _KP
_NW = 32
_CH = 128


def _sc_gather(tbl, idx):
    B = idx.shape[0]
    D = tbl.shape[1]
    bpw = B // _NW
    mesh = plsc.VectorSubcoreMesh(core_axis_name="c", subcore_axis_name="s")

    @functools.partial(
        pl.kernel,
        mesh=mesh,
        out_type=jax.ShapeDtypeStruct((B, D), jnp.float32),
        scratch_types=[
            pltpu.VMEM((_CH,), jnp.int32),
            pltpu.VMEM((_CH, D), jnp.float32),
            pltpu.SemaphoreType.DMA,
        ],
    )
    def k(tbl_hbm, idx_hbm, out_hbm, idx_v, rows_v, sem):
        wid = lax.axis_index("s") * 2 + lax.axis_index("c")
        base = wid * bpw
        for ci in range(bpw // _CH):
            off = base + ci * _CH
            pltpu.sync_copy(idx_hbm.at[pl.ds(off, _CH)], idx_v)
            pltpu.async_copy(tbl_hbm.at[idx_v], rows_v, sem).wait()
            pltpu.sync_copy(rows_v, out_hbm.at[pl.ds(off, _CH)])

    return k(tbl, idx)


def _mlp_body(g_ref, m_ref, sid_ref, semb_ref,
              w1_ref, b1_ref, w2_ref, b2_ref, w3_ref, b3_ref,
              out_ref, ekp_ref):
    bb = m_ref.shape[0]
    stu_n = semb_ref.shape[0]
    ids = sid_ref[...]
    row = lax.broadcasted_iota(jnp.int32, (bb, stu_n), 1)
    oh = (ids == row).astype(jnp.float32)
    stu = jnp.dot(oh, semb_ref[...], preferred_element_type=jnp.float32)
    stat = jax.nn.sigmoid(stu)

    kd = jax.nn.sigmoid(g_ref[:, :_K])
    ekp = g_ref[:, _KP:_KP + _K]
    ekp_ref[...] = ekp
    disc = jax.nn.sigmoid(g_ref[:, _DISC_COL:_DISC_COL + 1]) * 10.0

    x = disc * (stat - kd) * (m_ref[...] * jax.nn.sigmoid(ekp))
    h1 = jax.nn.sigmoid(
        jnp.dot(x, w1_ref[...], preferred_element_type=jnp.float32)
        + b1_ref[...])
    h2 = jax.nn.sigmoid(
        jnp.dot(h1, w2_ref[...], preferred_element_type=jnp.float32)
        + b2_ref[...])
    p = jax.nn.sigmoid(
        jnp.dot(h2, w3_ref[...], preferred_element_type=jnp.float32)
        + b3_ref[...])
    out_ref[:, 0:1] = 1.0 - p
    out_ref[:, 1:2] = p


def _tc_mlp(gathered, masks, sid2, student_emb, w1t, b1r, w2t, b2r, w3t, b3r):
    B = masks.shape[0]
    BB = 1024
    grid = (B // BB,)
    stu_n, k = student_emb.shape
    l1 = w1t.shape[1]
    l2 = w2t.shape[1]
    full = lambda shp: pl.BlockSpec(shp, lambda i: (0, 0))
    return pl.pallas_call(
        _mlp_body,
        grid=grid,
        in_specs=[
            pl.BlockSpec((BB, _D), lambda i: (i, 0)),
            pl.BlockSpec((BB, k), lambda i: (i, 0)),
            pl.BlockSpec((BB, 1), lambda i: (i, 0)),
            full((stu_n, k)),
            full((k, l1)), full((1, l1)),
            full((l1, l2)), full((1, l2)),
            full((l2, 1)), full((1, 1)),
        ],
        out_specs=[
            pl.BlockSpec((BB, 2), lambda i: (i, 0)),
            pl.BlockSpec((BB, k), lambda i: (i, 0)),
        ],
        out_shape=[
            jax.ShapeDtypeStruct((B, 2), jnp.float32),
            jax.ShapeDtypeStruct((B, k), jnp.float32),
        ],
    )(gathered, masks, sid2, student_emb,
      w1t, b1r, w2t, b2r, w3t, b3r)


def kernel(stu_id, input_exercise, knowledge_masks, student_emb, k_difficulty,
           e_discrimination, e_k_prob, W1, b1, W2, b2, W3, b3):
    exer_n, k = k_difficulty.shape
    z = jnp.zeros((exer_n, _KP - k), jnp.float32)
    tbl = jnp.concatenate(
        [k_difficulty, z, e_k_prob, e_discrimination,
         jnp.zeros((exer_n, _D - _DISC_COL - 1), jnp.float32)],
        axis=1)

    gathered = _sc_gather(tbl, input_exercise.astype(jnp.int32))

    out, ekp = _tc_mlp(
        gathered, knowledge_masks, stu_id.astype(jnp.int32).reshape(-1, 1),
        student_emb, W1.T, b1.reshape(1, -1), W2.T, b2.reshape(1, -1),
        W3.T, b3.reshape(1, -1))
    return (out, ekp)
